# Initial kernel scaffold; baseline (speedup 1.0000x reference)
import jax
import jax.numpy as jnp
from jax.experimental import pallas as pl


def kernel(batch, atom_type, pos_feat, charge, emb_atom, W1, b1, W2, b2, emb_charge, W_proj):
    raise NotImplementedError("write your pallas kernel here")


# speedup vs baseline: 2.1355x; 2.1355x over previous
import functools

import jax
import jax.numpy as jnp
from jax import lax
from jax.experimental import pallas as pl
from jax.experimental.pallas import tpu as pltpu
from jax.experimental.pallas import tpu_sc as plsc

N = 100000
NC = 2
NS = 16
NW = NC * NS
B_PER_W = 3200
NPAD = NW * B_PER_W
CHUNK = 128
NCHUNK = B_PER_W // CHUNK
BN = 512


def _sc_gather(emb_atom, atom_idx, batch_idx, charge_i):
    E1 = emb_atom.shape[1]
    G = charge_i.shape[0]
    mesh = plsc.VectorSubcoreMesh(core_axis_name="c", subcore_axis_name="s")

    @functools.partial(
        pl.kernel,
        out_type=(
            jax.ShapeDtypeStruct((NPAD, E1), jnp.float32),
            jax.ShapeDtypeStruct((NPAD,), jnp.int32),
        ),
        mesh=mesh,
        compiler_params=pltpu.CompilerParams(use_tc_tiling_on_sc=False),
        scratch_types=[
            pltpu.VMEM((NCHUNK, CHUNK), jnp.int32),
            pltpu.VMEM((NCHUNK, CHUNK), jnp.int32),
            pltpu.VMEM((B_PER_W,), jnp.int32),
            pltpu.VMEM((CHUNK, E1), jnp.float32),
            pltpu.SemaphoreType.DMA,
            pltpu.SemaphoreType.DMA,
        ],
    )
    def k(table_hbm, idx_hbm, batch_hbm, charge_hbm, ea_hbm, cpn_hbm,
          idx_v, batch_v, cpn_v, rows_v, gsem, csem):
        wid = lax.axis_index("s") * NC + lax.axis_index("c")
        base = wid * B_PER_W
        pltpu.sync_copy(idx_hbm.at[wid], idx_v)
        pltpu.sync_copy(batch_hbm.at[wid], batch_v)

        def row_body(j, carry):
            cp = pltpu.async_copy(
                charge_hbm.at[batch_v.at[j]], cpn_v.at[pl.ds(j * CHUNK, CHUNK)], csem)
            pltpu.async_copy(table_hbm.at[idx_v.at[j]], rows_v, gsem).wait()
            pltpu.sync_copy(rows_v, ea_hbm.at[pl.ds(base + j * CHUNK, CHUNK)])
            cp.wait()
            return carry

        lax.fori_loop(0, NCHUNK, row_body, 0)
        pltpu.sync_copy(cpn_v, cpn_hbm.at[pl.ds(base, B_PER_W)])

    return k(emb_atom, atom_idx, batch_idx, charge_i)


def _tc_fused(pos_feat, ea_pad, cpn3, W1, b1r, W2, b2r, Wp_a, Wp_h, emb_charge, Wp_c):
    IN = pos_feat.shape[1]
    E1 = ea_pad.shape[1]
    VC, E3 = emb_charge.shape
    OUT = Wp_a.shape[1]
    nb = pl.cdiv(N, BN)

    def body(pf_ref, ea_ref, cpn_ref, w1_ref, b1_ref, w2_ref, b2_ref,
             wpa_ref, wph_ref, ec_ref, wpc_ref, out_ref):
        h1 = jnp.dot(pf_ref[...], w1_ref[...], preferred_element_type=jnp.float32)
        h1 = h1 + b1_ref[...]
        h1 = h1 * jax.nn.sigmoid(h1)
        h = jnp.dot(h1, w2_ref[...], preferred_element_type=jnp.float32) + b2_ref[...]
        acc = jnp.dot(ea_ref[...], wpa_ref[...], preferred_element_type=jnp.float32)
        acc = acc + jnp.dot(h, wph_ref[...], preferred_element_type=jnp.float32)
        cg = jnp.dot(ec_ref[...], wpc_ref[...], preferred_element_type=jnp.float32)
        cpn = cpn_ref[0, 0, :]
        oh = (cpn[:, None] == lax.broadcasted_iota(jnp.int32, (BN, VC), 1)
              ).astype(jnp.float32)
        acc = acc + jnp.dot(oh, cg, preferred_element_type=jnp.float32)
        out_ref[...] = acc * jax.nn.sigmoid(acc)

    rep = lambda i: (0, 0)
    return pl.pallas_call(
        body,
        grid=(nb,),
        in_specs=[
            pl.BlockSpec((BN, IN), lambda i: (i, 0)),
            pl.BlockSpec((BN, E1), lambda i: (i, 0)),
            pl.BlockSpec((1, 1, BN), lambda i: (i, 0, 0)),
            pl.BlockSpec((IN, E1), rep),
            pl.BlockSpec((1, E1), rep),
            pl.BlockSpec((E1, E1), rep),
            pl.BlockSpec((1, E1), rep),
            pl.BlockSpec((E1, OUT), rep),
            pl.BlockSpec((E1, OUT), rep),
            pl.BlockSpec((VC, E3), rep),
            pl.BlockSpec((E3, OUT), rep),
        ],
        out_specs=pl.BlockSpec((BN, OUT), lambda i: (i, 0)),
        out_shape=jax.ShapeDtypeStruct((N, OUT), jnp.float32),
    )(pos_feat, ea_pad, cpn3, W1, b1r, W2, b2r, Wp_a, Wp_h, emb_charge, Wp_c)


def kernel(batch, atom_type, pos_feat, charge, emb_atom, W1, b1, W2, b2, emb_charge, W_proj):
    E1 = emb_atom.shape[1]
    E2 = W2.shape[1]
    pad = NPAD - N
    atom_idx = jnp.pad(atom_type.astype(jnp.int32), (0, pad)).reshape(NW, NCHUNK, CHUNK)
    batch_idx = jnp.pad(batch.astype(jnp.int32), (0, pad)).reshape(NW, NCHUNK, CHUNK)
    ea_pad, cpn = _sc_gather(emb_atom, atom_idx, batch_idx, charge.astype(jnp.int32))
    cpn3 = cpn.reshape(NPAD // BN, 1, BN)
    Wp_a = W_proj[:E1]
    Wp_h = W_proj[E1:E1 + E2]
    Wp_c = W_proj[E1 + E2:]
    return _tc_fused(pos_feat, ea_pad, cpn3, W1, b1.reshape(1, -1), W2,
                     b2.reshape(1, -1), Wp_a, Wp_h, emb_charge, Wp_c)


# speedup vs baseline: 2.1643x; 1.0135x over previous
import functools

import jax
import jax.numpy as jnp
from jax import lax
from jax.experimental import pallas as pl
from jax.experimental.pallas import tpu as pltpu
from jax.experimental.pallas import tpu_sc as plsc

N = 100000
NC = 2
NS = 16
NW = NC * NS
B_PER_W = 3200
NPAD = NW * B_PER_W
CHUNK = 128
NCHUNK = B_PER_W // CHUNK
BN = 512


def _sc_gather(emb_atom, atom_idx, batch_idx, charge_i):
    E1 = emb_atom.shape[1]
    G = charge_i.shape[0]
    mesh = plsc.VectorSubcoreMesh(core_axis_name="c", subcore_axis_name="s")

    @functools.partial(
        pl.kernel,
        out_type=(
            jax.ShapeDtypeStruct((NPAD, E1), jnp.float32),
            jax.ShapeDtypeStruct((NPAD,), jnp.int32),
        ),
        mesh=mesh,
        compiler_params=pltpu.CompilerParams(use_tc_tiling_on_sc=False),
        scratch_types=[
            pltpu.VMEM((B_PER_W,), jnp.int32),
            pltpu.VMEM((B_PER_W,), jnp.int32),
            pltpu.VMEM((B_PER_W,), jnp.int32),
            pltpu.VMEM((CHUNK, E1), jnp.float32),
            pltpu.VMEM((CHUNK, E1), jnp.float32),
            pltpu.VMEM((CHUNK, E1), jnp.float32),
            pltpu.VMEM((CHUNK, E1), jnp.float32),
            pltpu.VMEM((CHUNK, E1), jnp.float32),
            pltpu.SemaphoreType.DMA,
            pltpu.SemaphoreType.DMA,
            pltpu.SemaphoreType.DMA,
            pltpu.SemaphoreType.DMA,
            pltpu.SemaphoreType.DMA,
            pltpu.SemaphoreType.DMA,
            pltpu.SemaphoreType.DMA,
        ],
    )
    def k(table_hbm, idx_hbm, batch_hbm, charge_hbm, ea_hbm, cpn_hbm,
          idx_v, batch_v, cpn_v, r0, r1, r2, r3, r4, g0, g1, g2, g3, g4,
          wsem, csem):
        wid = lax.axis_index("s") * NC + lax.axis_index("c")
        base = wid * B_PER_W
        pltpu.sync_copy(idx_hbm.at[pl.ds(base, B_PER_W)], idx_v)
        pltpu.sync_copy(batch_hbm.at[pl.ds(base, B_PER_W)], batch_v)

        rows = (r0, r1, r2, r3, r4)
        gsems = (g0, g1, g2, g3, g4)
        GRP = 5

        def body(i, carry):
            j0 = i * GRP
            cd = [pltpu.async_copy(
                charge_hbm.at[batch_v.at[pl.ds((j0 + k) * CHUNK, CHUNK)]],
                cpn_v.at[pl.ds((j0 + k) * CHUNK, CHUNK)], csem)
                for k in range(GRP)]
            gd = [pltpu.async_copy(
                table_hbm.at[idx_v.at[pl.ds((j0 + k) * CHUNK, CHUNK)]],
                rows[k], gsems[k])
                for k in range(GRP)]
            wd = []
            for k in range(GRP):
                gd[k].wait()
                wd.append(pltpu.async_copy(
                    rows[k], ea_hbm.at[pl.ds(base + (j0 + k) * CHUNK, CHUNK)],
                    wsem))
            for k in range(GRP):
                wd[k].wait()
                cd[k].wait()
            return carry

        lax.fori_loop(0, NCHUNK // GRP, body, 0)
        pltpu.sync_copy(cpn_v, cpn_hbm.at[pl.ds(base, B_PER_W)])

    return k(emb_atom, atom_idx, batch_idx, charge_i)


def _tc_fused(pos_feat, ea_pad, cpn3, W1, b1r, W2, b2r, Wp_a, Wp_h, emb_charge, Wp_c):
    IN = pos_feat.shape[1]
    E1 = ea_pad.shape[1]
    VC, E3 = emb_charge.shape
    OUT = Wp_a.shape[1]
    nb = pl.cdiv(N, BN)

    def body(pf_ref, ea_ref, cpn_ref, w1_ref, b1_ref, w2_ref, b2_ref,
             wpa_ref, wph_ref, ec_ref, wpc_ref, out_ref):
        h1 = jnp.dot(pf_ref[...], w1_ref[...], preferred_element_type=jnp.float32)
        h1 = h1 + b1_ref[...]
        h1 = h1 * jax.nn.sigmoid(h1)
        h = jnp.dot(h1, w2_ref[...], preferred_element_type=jnp.float32) + b2_ref[...]
        acc = jnp.dot(ea_ref[...], wpa_ref[...], preferred_element_type=jnp.float32)
        acc = acc + jnp.dot(h, wph_ref[...], preferred_element_type=jnp.float32)
        cg = jnp.dot(ec_ref[...], wpc_ref[...], preferred_element_type=jnp.float32)
        cpn = cpn_ref[0, 0, :]
        oh = (cpn[:, None] == lax.broadcasted_iota(jnp.int32, (BN, VC), 1)
              ).astype(jnp.float32)
        acc = acc + jnp.dot(oh, cg, preferred_element_type=jnp.float32)
        out_ref[...] = acc * jax.nn.sigmoid(acc)

    rep = lambda i: (0, 0)
    return pl.pallas_call(
        body,
        grid=(nb,),
        in_specs=[
            pl.BlockSpec((BN, IN), lambda i: (i, 0)),
            pl.BlockSpec((BN, E1), lambda i: (i, 0)),
            pl.BlockSpec((1, 1, BN), lambda i: (i, 0, 0)),
            pl.BlockSpec((IN, E1), rep),
            pl.BlockSpec((1, E1), rep),
            pl.BlockSpec((E1, E1), rep),
            pl.BlockSpec((1, E1), rep),
            pl.BlockSpec((E1, OUT), rep),
            pl.BlockSpec((E1, OUT), rep),
            pl.BlockSpec((VC, E3), rep),
            pl.BlockSpec((E3, OUT), rep),
        ],
        out_specs=pl.BlockSpec((BN, OUT), lambda i: (i, 0)),
        out_shape=jax.ShapeDtypeStruct((N, OUT), jnp.float32),
    )(pos_feat, ea_pad, cpn3, W1, b1r, W2, b2r, Wp_a, Wp_h, emb_charge, Wp_c)


def kernel(batch, atom_type, pos_feat, charge, emb_atom, W1, b1, W2, b2, emb_charge, W_proj):
    E1 = emb_atom.shape[1]
    E2 = W2.shape[1]
    pad = NPAD - N
    atom_idx = jnp.pad(atom_type.astype(jnp.int32), (0, pad))
    batch_idx = jnp.pad(batch.astype(jnp.int32), (0, pad))
    ea_pad, cpn = _sc_gather(emb_atom, atom_idx, batch_idx, charge.astype(jnp.int32))
    cpn3 = cpn.reshape(NPAD // BN, 1, BN)
    Wp_a = W_proj[:E1]
    Wp_h = W_proj[E1:E1 + E2]
    Wp_c = W_proj[E1 + E2:]
    return _tc_fused(pos_feat, ea_pad, cpn3, W1, b1.reshape(1, -1), W2,
                     b2.reshape(1, -1), Wp_a, Wp_h, emb_charge, Wp_c)


# speedup vs baseline: 2.5955x; 1.1992x over previous
import functools

import jax
import jax.numpy as jnp
from jax import lax
from jax.experimental import pallas as pl
from jax.experimental.pallas import tpu as pltpu
from jax.experimental.pallas import tpu_sc as plsc

N = 100000
NC = 2
NS = 16
NW = NC * NS
B_PER_W = 3200
NPAD = NW * B_PER_W
CHUNK = 128
NCHUNK = B_PER_W // CHUNK
BN = 1024


def _sc_gather(emb_atom, atom_idx, batch_idx, charge_i):
    E1 = emb_atom.shape[1]
    G = charge_i.shape[0]
    mesh = plsc.VectorSubcoreMesh(core_axis_name="c", subcore_axis_name="s")

    @functools.partial(
        pl.kernel,
        out_type=(
            jax.ShapeDtypeStruct((NPAD, E1), jnp.float32),
            jax.ShapeDtypeStruct((NPAD,), jnp.int32),
        ),
        mesh=mesh,
        compiler_params=pltpu.CompilerParams(use_tc_tiling_on_sc=False),
        scratch_types=[
            pltpu.VMEM((B_PER_W,), jnp.int32),
            pltpu.VMEM((B_PER_W,), jnp.int32),
            pltpu.VMEM((B_PER_W,), jnp.int32),
            pltpu.VMEM((CHUNK, E1), jnp.float32),
            pltpu.VMEM((CHUNK, E1), jnp.float32),
            pltpu.VMEM((CHUNK, E1), jnp.float32),
            pltpu.VMEM((CHUNK, E1), jnp.float32),
            pltpu.VMEM((CHUNK, E1), jnp.float32),
            pltpu.SemaphoreType.DMA,
            pltpu.SemaphoreType.DMA,
            pltpu.SemaphoreType.DMA,
            pltpu.SemaphoreType.DMA,
            pltpu.SemaphoreType.DMA,
            pltpu.SemaphoreType.DMA,
            pltpu.SemaphoreType.DMA,
        ],
    )
    def k(table_hbm, idx_hbm, batch_hbm, charge_hbm, ea_hbm, cpn_hbm,
          idx_v, batch_v, cpn_v, r0, r1, r2, r3, r4, g0, g1, g2, g3, g4,
          wsem, csem):
        wid = lax.axis_index("s") * NC + lax.axis_index("c")
        base = wid * B_PER_W
        pltpu.sync_copy(idx_hbm.at[pl.ds(base, B_PER_W)], idx_v)
        pltpu.sync_copy(batch_hbm.at[pl.ds(base, B_PER_W)], batch_v)

        rows = (r0, r1, r2, r3, r4)
        gsems = (g0, g1, g2, g3, g4)
        GRP = 5

        def body(i, carry):
            j0 = i * GRP
            cd = [pltpu.async_copy(
                charge_hbm.at[batch_v.at[pl.ds((j0 + k) * CHUNK, CHUNK)]],
                cpn_v.at[pl.ds((j0 + k) * CHUNK, CHUNK)], csem)
                for k in range(GRP)]
            gd = [pltpu.async_copy(
                table_hbm.at[idx_v.at[pl.ds((j0 + k) * CHUNK, CHUNK)]],
                rows[k], gsems[k])
                for k in range(GRP)]
            wd = []
            for k in range(GRP):
                gd[k].wait()
                wd.append(pltpu.async_copy(
                    rows[k], ea_hbm.at[pl.ds(base + (j0 + k) * CHUNK, CHUNK)],
                    wsem))
            for k in range(GRP):
                wd[k].wait()
                cd[k].wait()
            return carry

        lax.fori_loop(0, NCHUNK // GRP, body, 0)
        pltpu.sync_copy(cpn_v, cpn_hbm.at[pl.ds(base, B_PER_W)])

    return k(emb_atom, atom_idx, batch_idx, charge_i)


def _tc_fused(pos_feat, ea_pad, cpn3, W1, b1r, W2, b2r, Wp_a, Wp_h, emb_charge, Wp_c):
    IN = pos_feat.shape[1]
    E1 = ea_pad.shape[1]
    VC, E3 = emb_charge.shape
    OUT = Wp_a.shape[1]
    nb = pl.cdiv(N, BN)

    def body(pf_ref, ea_ref, cpn_ref, w1_ref, b1_ref, w2_ref, b2_ref,
             wpa_ref, wph_ref, ec_ref, wpc_ref, out_ref):
        h1 = jnp.dot(pf_ref[...], w1_ref[...], preferred_element_type=jnp.float32)
        h1 = h1 + b1_ref[...]
        h1 = h1 * jax.nn.sigmoid(h1)
        h = jnp.dot(h1, w2_ref[...], preferred_element_type=jnp.float32) + b2_ref[...]
        acc = jnp.dot(ea_ref[...], wpa_ref[...], preferred_element_type=jnp.float32)
        acc = acc + jnp.dot(h, wph_ref[...], preferred_element_type=jnp.float32)
        cg = jnp.dot(ec_ref[...], wpc_ref[...], preferred_element_type=jnp.float32)
        cpn = cpn_ref[0, 0, :]
        oh = (cpn[:, None] == lax.broadcasted_iota(jnp.int32, (BN, VC), 1)
              ).astype(jnp.float32)
        acc = acc + jnp.dot(oh, cg, preferred_element_type=jnp.float32)
        out_ref[...] = acc * jax.nn.sigmoid(acc)

    rep = lambda i: (0, 0)
    return pl.pallas_call(
        body,
        grid=(nb,),
        in_specs=[
            pl.BlockSpec((BN, IN), lambda i: (i, 0)),
            pl.BlockSpec((BN, E1), lambda i: (i, 0)),
            pl.BlockSpec((1, 1, BN), lambda i: (i, 0, 0)),
            pl.BlockSpec((IN, E1), rep),
            pl.BlockSpec((1, E1), rep),
            pl.BlockSpec((E1, E1), rep),
            pl.BlockSpec((1, E1), rep),
            pl.BlockSpec((E1, OUT), rep),
            pl.BlockSpec((E1, OUT), rep),
            pl.BlockSpec((VC, E3), rep),
            pl.BlockSpec((E3, OUT), rep),
        ],
        out_specs=pl.BlockSpec((BN, OUT), lambda i: (i, 0)),
        out_shape=jax.ShapeDtypeStruct((N, OUT), jnp.float32),
    )(pos_feat, ea_pad, cpn3, W1, b1r, W2, b2r, Wp_a, Wp_h, emb_charge, Wp_c)


def kernel(batch, atom_type, pos_feat, charge, emb_atom, W1, b1, W2, b2, emb_charge, W_proj):
    E1 = emb_atom.shape[1]
    E2 = W2.shape[1]
    pad = NPAD - N
    atom_idx = jnp.pad(atom_type.astype(jnp.int32), (0, pad))
    batch_idx = jnp.pad(batch.astype(jnp.int32), (0, pad))
    ea_pad, cpn = _sc_gather(emb_atom, atom_idx, batch_idx, charge.astype(jnp.int32))
    cpn3 = cpn.reshape(NPAD // BN, 1, BN)
    Wp_a = W_proj[:E1]
    Wp_h = W_proj[E1:E1 + E2]
    Wp_c = W_proj[E1 + E2:]
    return _tc_fused(pos_feat, ea_pad, cpn3, W1, b1.reshape(1, -1), W2,
                     b2.reshape(1, -1), Wp_a, Wp_h, emb_charge, Wp_c)


# speedup vs baseline: 2.8424x; 1.0951x over previous
import functools

import jax
import jax.numpy as jnp
from jax import lax
from jax.experimental import pallas as pl
from jax.experimental.pallas import tpu as pltpu
from jax.experimental.pallas import tpu_sc as plsc

N = 100000
NC = 2
NS = 16
NW = NC * NS
B_PER_W = 3200
NPAD = NW * B_PER_W
CHUNK = 128
NCHUNK = B_PER_W // CHUNK
BN = 2048


def _sc_gather(emb_atom, atom_idx, batch_idx, charge_i):
    E1 = emb_atom.shape[1]
    G = charge_i.shape[0]
    mesh = plsc.VectorSubcoreMesh(core_axis_name="c", subcore_axis_name="s")

    @functools.partial(
        pl.kernel,
        out_type=(
            jax.ShapeDtypeStruct((NPAD, E1), jnp.float32),
            jax.ShapeDtypeStruct((NPAD,), jnp.int32),
        ),
        mesh=mesh,
        compiler_params=pltpu.CompilerParams(use_tc_tiling_on_sc=False),
        scratch_types=[
            pltpu.VMEM((B_PER_W,), jnp.int32),
            pltpu.VMEM((B_PER_W,), jnp.int32),
            pltpu.VMEM((B_PER_W,), jnp.int32),
            pltpu.VMEM((CHUNK, E1), jnp.float32),
            pltpu.VMEM((CHUNK, E1), jnp.float32),
            pltpu.VMEM((CHUNK, E1), jnp.float32),
            pltpu.VMEM((CHUNK, E1), jnp.float32),
            pltpu.VMEM((CHUNK, E1), jnp.float32),
            pltpu.SemaphoreType.DMA,
            pltpu.SemaphoreType.DMA,
            pltpu.SemaphoreType.DMA,
            pltpu.SemaphoreType.DMA,
            pltpu.SemaphoreType.DMA,
            pltpu.SemaphoreType.DMA,
            pltpu.SemaphoreType.DMA,
        ],
    )
    def k(table_hbm, idx_hbm, batch_hbm, charge_hbm, ea_hbm, cpn_hbm,
          idx_v, batch_v, cpn_v, r0, r1, r2, r3, r4, g0, g1, g2, g3, g4,
          wsem, csem):
        wid = lax.axis_index("s") * NC + lax.axis_index("c")
        base = wid * B_PER_W
        pltpu.sync_copy(idx_hbm.at[pl.ds(base, B_PER_W)], idx_v)
        pltpu.sync_copy(batch_hbm.at[pl.ds(base, B_PER_W)], batch_v)

        rows = (r0, r1, r2, r3, r4)
        gsems = (g0, g1, g2, g3, g4)
        GRP = 5

        def body(i, carry):
            j0 = i * GRP
            cd = [pltpu.async_copy(
                charge_hbm.at[batch_v.at[pl.ds((j0 + k) * CHUNK, CHUNK)]],
                cpn_v.at[pl.ds((j0 + k) * CHUNK, CHUNK)], csem)
                for k in range(GRP)]
            gd = [pltpu.async_copy(
                table_hbm.at[idx_v.at[pl.ds((j0 + k) * CHUNK, CHUNK)]],
                rows[k], gsems[k])
                for k in range(GRP)]
            wd = []
            for k in range(GRP):
                gd[k].wait()
                wd.append(pltpu.async_copy(
                    rows[k], ea_hbm.at[pl.ds(base + (j0 + k) * CHUNK, CHUNK)],
                    wsem))
            for k in range(GRP):
                wd[k].wait()
                cd[k].wait()
            return carry

        lax.fori_loop(0, NCHUNK // GRP, body, 0)
        pltpu.sync_copy(cpn_v, cpn_hbm.at[pl.ds(base, B_PER_W)])

    return k(emb_atom, atom_idx, batch_idx, charge_i)


def _tc_fused(pos_feat, ea_pad, cpn3, W1, b1r, W2, b2r, Wp_a, Wp_h, emb_charge, Wp_c):
    IN = pos_feat.shape[1]
    E1 = ea_pad.shape[1]
    VC, E3 = emb_charge.shape
    OUT = Wp_a.shape[1]
    nb = pl.cdiv(N, BN)

    def body(pf_ref, ea_ref, cpn_ref, w1_ref, b1_ref, w2_ref, b2_ref,
             wpa_ref, wph_ref, ec_ref, wpc_ref, out_ref):
        h1 = jnp.dot(pf_ref[...], w1_ref[...], preferred_element_type=jnp.float32)
        h1 = h1 + b1_ref[...]
        h1 = h1 * jax.nn.sigmoid(h1)
        h = jnp.dot(h1, w2_ref[...], preferred_element_type=jnp.float32) + b2_ref[...]
        acc = jnp.dot(ea_ref[...], wpa_ref[...], preferred_element_type=jnp.float32)
        acc = acc + jnp.dot(h, wph_ref[...], preferred_element_type=jnp.float32)
        cg = jnp.dot(ec_ref[...], wpc_ref[...], preferred_element_type=jnp.float32)
        cpn = cpn_ref[0, 0, :]
        oh = (cpn[:, None] == lax.broadcasted_iota(jnp.int32, (BN, VC), 1)
              ).astype(jnp.float32)
        acc = acc + jnp.dot(oh, cg, preferred_element_type=jnp.float32)
        out_ref[...] = acc * jax.nn.sigmoid(acc)

    rep = lambda i: (0, 0)
    return pl.pallas_call(
        body,
        grid=(nb,),
        in_specs=[
            pl.BlockSpec((BN, IN), lambda i: (i, 0)),
            pl.BlockSpec((BN, E1), lambda i: (i, 0)),
            pl.BlockSpec((1, 1, BN), lambda i: (i, 0, 0)),
            pl.BlockSpec((IN, E1), rep),
            pl.BlockSpec((1, E1), rep),
            pl.BlockSpec((E1, E1), rep),
            pl.BlockSpec((1, E1), rep),
            pl.BlockSpec((E1, OUT), rep),
            pl.BlockSpec((E1, OUT), rep),
            pl.BlockSpec((VC, E3), rep),
            pl.BlockSpec((E3, OUT), rep),
        ],
        out_specs=pl.BlockSpec((BN, OUT), lambda i: (i, 0)),
        out_shape=jax.ShapeDtypeStruct((N, OUT), jnp.float32),
    )(pos_feat, ea_pad, cpn3, W1, b1r, W2, b2r, Wp_a, Wp_h, emb_charge, Wp_c)


def kernel(batch, atom_type, pos_feat, charge, emb_atom, W1, b1, W2, b2, emb_charge, W_proj):
    E1 = emb_atom.shape[1]
    E2 = W2.shape[1]
    pad = NPAD - N
    atom_idx = jnp.pad(atom_type.astype(jnp.int32), (0, pad))
    batch_idx = jnp.pad(batch.astype(jnp.int32), (0, pad))
    ea_pad, cpn = _sc_gather(emb_atom, atom_idx, batch_idx, charge.astype(jnp.int32))
    cpn3 = cpn.reshape(NPAD // BN, 1, BN)
    Wp_a = W_proj[:E1]
    Wp_h = W_proj[E1:E1 + E2]
    Wp_c = W_proj[E1 + E2:]
    return _tc_fused(pos_feat, ea_pad, cpn3, W1, b1.reshape(1, -1), W2,
                     b2.reshape(1, -1), Wp_a, Wp_h, emb_charge, Wp_c)


# speedup vs baseline: 3.0049x; 1.0572x over previous
import functools

import jax
import jax.numpy as jnp
from jax import lax
from jax.experimental import pallas as pl
from jax.experimental.pallas import tpu as pltpu
from jax.experimental.pallas import tpu_sc as plsc

N = 100000
NC = 2
NS = 16
NW = NC * NS
B_PER_W = 3200
NPAD = NW * B_PER_W
CHUNK = 128
NCHUNK = B_PER_W // CHUNK
BN = 4096


def _sc_gather(emb_atom, atom_idx, batch_idx, charge_i):
    E1 = emb_atom.shape[1]
    G = charge_i.shape[0]
    mesh = plsc.VectorSubcoreMesh(core_axis_name="c", subcore_axis_name="s")

    @functools.partial(
        pl.kernel,
        out_type=(
            jax.ShapeDtypeStruct((NPAD, E1), jnp.float32),
            jax.ShapeDtypeStruct((NPAD,), jnp.int32),
        ),
        mesh=mesh,
        compiler_params=pltpu.CompilerParams(use_tc_tiling_on_sc=False),
        scratch_types=[
            pltpu.VMEM((B_PER_W,), jnp.int32),
            pltpu.VMEM((B_PER_W,), jnp.int32),
            pltpu.VMEM((B_PER_W,), jnp.int32),
            pltpu.VMEM((CHUNK, E1), jnp.float32),
            pltpu.VMEM((CHUNK, E1), jnp.float32),
            pltpu.VMEM((CHUNK, E1), jnp.float32),
            pltpu.VMEM((CHUNK, E1), jnp.float32),
            pltpu.VMEM((CHUNK, E1), jnp.float32),
            pltpu.SemaphoreType.DMA,
            pltpu.SemaphoreType.DMA,
            pltpu.SemaphoreType.DMA,
            pltpu.SemaphoreType.DMA,
            pltpu.SemaphoreType.DMA,
            pltpu.SemaphoreType.DMA,
            pltpu.SemaphoreType.DMA,
        ],
    )
    def k(table_hbm, idx_hbm, batch_hbm, charge_hbm, ea_hbm, cpn_hbm,
          idx_v, batch_v, cpn_v, r0, r1, r2, r3, r4, g0, g1, g2, g3, g4,
          wsem, csem):
        wid = lax.axis_index("s") * NC + lax.axis_index("c")
        base = wid * B_PER_W
        pltpu.sync_copy(idx_hbm.at[pl.ds(base, B_PER_W)], idx_v)
        pltpu.sync_copy(batch_hbm.at[pl.ds(base, B_PER_W)], batch_v)

        rows = (r0, r1, r2, r3, r4)
        gsems = (g0, g1, g2, g3, g4)
        GRP = 5

        def body(i, carry):
            j0 = i * GRP
            cd = [pltpu.async_copy(
                charge_hbm.at[batch_v.at[pl.ds((j0 + k) * CHUNK, CHUNK)]],
                cpn_v.at[pl.ds((j0 + k) * CHUNK, CHUNK)], csem)
                for k in range(GRP)]
            gd = [pltpu.async_copy(
                table_hbm.at[idx_v.at[pl.ds((j0 + k) * CHUNK, CHUNK)]],
                rows[k], gsems[k])
                for k in range(GRP)]
            wd = []
            for k in range(GRP):
                gd[k].wait()
                wd.append(pltpu.async_copy(
                    rows[k], ea_hbm.at[pl.ds(base + (j0 + k) * CHUNK, CHUNK)],
                    wsem))
            for k in range(GRP):
                wd[k].wait()
                cd[k].wait()
            return carry

        lax.fori_loop(0, NCHUNK // GRP, body, 0)
        pltpu.sync_copy(cpn_v, cpn_hbm.at[pl.ds(base, B_PER_W)])

    return k(emb_atom, atom_idx, batch_idx, charge_i)


def _tc_fused(pos_feat, ea_pad, cpn3, W1, b1r, W2, b2r, Wp_a, Wp_h, emb_charge, Wp_c):
    IN = pos_feat.shape[1]
    E1 = ea_pad.shape[1]
    VC, E3 = emb_charge.shape
    OUT = Wp_a.shape[1]
    nb = pl.cdiv(N, BN)

    def body(pf_ref, ea_ref, cpn_ref, w1_ref, b1_ref, w2_ref, b2_ref,
             wpa_ref, wph_ref, ec_ref, wpc_ref, out_ref):
        h1 = jnp.dot(pf_ref[...], w1_ref[...], preferred_element_type=jnp.float32)
        h1 = h1 + b1_ref[...]
        h1 = h1 * jax.nn.sigmoid(h1)
        h = jnp.dot(h1, w2_ref[...], preferred_element_type=jnp.float32) + b2_ref[...]
        acc = jnp.dot(ea_ref[...], wpa_ref[...], preferred_element_type=jnp.float32)
        acc = acc + jnp.dot(h, wph_ref[...], preferred_element_type=jnp.float32)
        cg = jnp.dot(ec_ref[...], wpc_ref[...], preferred_element_type=jnp.float32)
        cpn = cpn_ref[0, 0, :]
        oh = (cpn[:, None] == lax.broadcasted_iota(jnp.int32, (BN, VC), 1)
              ).astype(jnp.float32)
        acc = acc + jnp.dot(oh, cg, preferred_element_type=jnp.float32)
        out_ref[...] = acc * jax.nn.sigmoid(acc)

    rep = lambda i: (0, 0)
    return pl.pallas_call(
        body,
        grid=(nb,),
        in_specs=[
            pl.BlockSpec((BN, IN), lambda i: (i, 0)),
            pl.BlockSpec((BN, E1), lambda i: (i, 0)),
            pl.BlockSpec((1, 1, BN), lambda i: (i, 0, 0)),
            pl.BlockSpec((IN, E1), rep),
            pl.BlockSpec((1, E1), rep),
            pl.BlockSpec((E1, E1), rep),
            pl.BlockSpec((1, E1), rep),
            pl.BlockSpec((E1, OUT), rep),
            pl.BlockSpec((E1, OUT), rep),
            pl.BlockSpec((VC, E3), rep),
            pl.BlockSpec((E3, OUT), rep),
        ],
        out_specs=pl.BlockSpec((BN, OUT), lambda i: (i, 0)),
        out_shape=jax.ShapeDtypeStruct((N, OUT), jnp.float32),
    )(pos_feat, ea_pad, cpn3, W1, b1r, W2, b2r, Wp_a, Wp_h, emb_charge, Wp_c)


def kernel(batch, atom_type, pos_feat, charge, emb_atom, W1, b1, W2, b2, emb_charge, W_proj):
    E1 = emb_atom.shape[1]
    E2 = W2.shape[1]
    pad = NPAD - N
    atom_idx = jnp.pad(atom_type.astype(jnp.int32), (0, pad))
    batch_idx = jnp.pad(batch.astype(jnp.int32), (0, pad))
    ea_pad, cpn = _sc_gather(emb_atom, atom_idx, batch_idx, charge.astype(jnp.int32))
    cpn3 = cpn.reshape(NPAD // BN, 1, BN)
    Wp_a = W_proj[:E1]
    Wp_h = W_proj[E1:E1 + E2]
    Wp_c = W_proj[E1 + E2:]
    return _tc_fused(pos_feat, ea_pad, cpn3, W1, b1.reshape(1, -1), W2,
                     b2.reshape(1, -1), Wp_a, Wp_h, emb_charge, Wp_c)
